# baseline (device time: 63406 ns/iter reference)
import jax
import jax.numpy as jnp
from jax import lax
from jax.experimental import pallas as pl
from jax.experimental.pallas import tpu as pltpu

N_DEV = 4


def kernel(A, B):
    m, k = A.shape
    k2, n = B.shape
    C = m // N_DEV
    H = n // 2

    def body(a_ref, b_ref, out_ref,
             cw_send, cw_recv, ccw_send, ccw_recv, ag_cw, ag_ccw,
             cw_ssem, cw_rsem, ccw_ssem, ccw_rsem,
             agcw_ssem, agcw_rsem, agccw_ssem, agccw_rsem):
        d = lax.axis_index("i")
        left = (d + N_DEV - 1) % N_DEV
        right = (d + 1) % N_DEV

        barrier_sem = pltpu.get_barrier_semaphore()
        for nbr in (left, right):
            pl.semaphore_signal(
                barrier_sem, inc=1,
                device_id=(nbr,), device_id_type=pl.DeviceIdType.MESH,
            )
        pl.semaphore_wait(barrier_sem, 2)

        f32 = jnp.float32
        bf16 = jnp.bfloat16

        def p_left(c):
            return jnp.dot(a_ref[pl.ds(c * C, C), :], b_ref[:, :H],
                           preferred_element_type=f32)

        def p_right(c):
            return jnp.dot(a_ref[pl.ds(c * C, C), :], b_ref[:, H:],
                           preferred_element_type=f32)

        rs_cw, rs_ccw = [], []
        for s in range(N_DEV - 1):
            rs_cw.append(pltpu.make_async_remote_copy(
                src_ref=cw_send.at[s], dst_ref=cw_recv.at[s],
                send_sem=cw_ssem.at[s], recv_sem=cw_rsem.at[s],
                device_id=(right,), device_id_type=pl.DeviceIdType.MESH,
            ))
            rs_ccw.append(pltpu.make_async_remote_copy(
                src_ref=ccw_send.at[s], dst_ref=ccw_recv.at[s],
                send_sem=ccw_ssem.at[s], recv_sem=ccw_rsem.at[s],
                device_id=(left,), device_id_type=pl.DeviceIdType.MESH,
            ))
        ag_cws, ag_ccws = [], []
        for h in range(N_DEV - 1):
            ag_cws.append(pltpu.make_async_remote_copy(
                src_ref=ag_cw.at[h], dst_ref=ag_cw.at[h + 1],
                send_sem=agcw_ssem.at[h], recv_sem=agcw_rsem.at[h],
                device_id=(right,), device_id_type=pl.DeviceIdType.MESH,
            ))
            ag_ccws.append(pltpu.make_async_remote_copy(
                src_ref=ag_ccw.at[h], dst_ref=ag_ccw.at[h + 1],
                send_sem=agccw_ssem.at[h], recv_sem=agccw_rsem.at[h],
                device_id=(left,), device_id_type=pl.DeviceIdType.MESH,
            ))

        cw_send[0, :, :] = p_left((d + N_DEV - 1) % N_DEV).astype(bf16)
        rs_cw[0].start()
        ccw_send[0, :, :] = p_right((d + 1) % N_DEV).astype(bf16)
        rs_ccw[0].start()

        for s in range(N_DEV - 2):
            pcw = p_left((d + 2 * N_DEV - 2 - s) % N_DEV)
            rs_cw[s].wait_recv()
            cw_send[s + 1, :, :] = (pcw + cw_recv[s].astype(f32)).astype(bf16)
            rs_cw[s + 1].start()
            pccw = p_right((d + 2 + s) % N_DEV)
            rs_ccw[s].wait_recv()
            ccw_send[s + 1, :, :] = (
                pccw + ccw_recv[s].astype(f32)).astype(bf16)
            rs_ccw[s + 1].start()

        rows = pl.ds(d * C, C)
        pfull = jnp.dot(a_ref[rows, :], b_ref[...],
                        preferred_element_type=f32)
        last = N_DEV - 2
        rs_cw[last].wait_recv()
        lh = jnp.maximum(pfull[:, :H] + cw_recv[last].astype(f32), 0.0)
        ag_cw[0, :, :] = lh.astype(bf16)
        ag_cws[0].start()
        rs_ccw[last].wait_recv()
        rh = jnp.maximum(pfull[:, H:] + ccw_recv[last].astype(f32), 0.0)
        ag_ccw[0, :, :] = rh.astype(bf16)
        ag_ccws[0].start()
        out_ref[rows, :H] = lh
        out_ref[rows, H:] = rh

        for h in range(N_DEV - 1):
            ag_cws[h].wait_recv()
            if h < N_DEV - 2:
                ag_cws[h + 1].start()
            ag_ccws[h].wait_recv()
            if h < N_DEV - 2:
                ag_ccws[h + 1].start()
            o_cw = (d + N_DEV - 1 - h) % N_DEV
            o_ccw = (d + 1 + h) % N_DEV
            out_ref[pl.ds(o_cw * C, C), :H] = ag_cw[h + 1].astype(f32)
            out_ref[pl.ds(o_ccw * C, C), H:] = ag_ccw[h + 1].astype(f32)

        for r in rs_cw + rs_ccw + ag_cws + ag_ccws:
            r.wait_send()

    return pl.pallas_call(
        body,
        out_shape=jax.ShapeDtypeStruct((m, n), jnp.float32),
        in_specs=[
            pl.BlockSpec(memory_space=pltpu.VMEM),
            pl.BlockSpec(memory_space=pltpu.VMEM),
        ],
        out_specs=pl.BlockSpec(memory_space=pltpu.VMEM),
        scratch_shapes=[
            pltpu.VMEM((N_DEV - 1, C, H), jnp.bfloat16),
            pltpu.VMEM((N_DEV - 1, C, H), jnp.bfloat16),
            pltpu.VMEM((N_DEV - 1, C, H), jnp.bfloat16),
            pltpu.VMEM((N_DEV - 1, C, H), jnp.bfloat16),
            pltpu.VMEM((N_DEV, C, H), jnp.bfloat16),
            pltpu.VMEM((N_DEV, C, H), jnp.bfloat16),
            pltpu.SemaphoreType.DMA((N_DEV - 1,)),
            pltpu.SemaphoreType.DMA((N_DEV - 1,)),
            pltpu.SemaphoreType.DMA((N_DEV - 1,)),
            pltpu.SemaphoreType.DMA((N_DEV - 1,)),
            pltpu.SemaphoreType.DMA((N_DEV - 1,)),
            pltpu.SemaphoreType.DMA((N_DEV - 1,)),
            pltpu.SemaphoreType.DMA((N_DEV - 1,)),
            pltpu.SemaphoreType.DMA((N_DEV - 1,)),
        ],
        compiler_params=pltpu.CompilerParams(collective_id=0),
    )(A, B)


# device time: 52704 ns/iter; 1.2031x vs baseline; 1.2031x over previous
import jax
import jax.numpy as jnp
from jax import lax
from jax.experimental import pallas as pl
from jax.experimental.pallas import tpu as pltpu

N_DEV = 4
NSUB = 2


def kernel(A, B):
    m, k = A.shape
    k2, n = B.shape
    C = m // N_DEV
    H = n // 2
    R = C // NSUB

    def body(a_ref, b_ref, out_ref,
             cw_send, cw_recv, ccw_send, ccw_recv, ag_cw, ag_ccw,
             cw_ssem, cw_rsem, ccw_ssem, ccw_rsem,
             agcw_ssem, agcw_rsem, agccw_ssem, agccw_rsem):
        d = lax.axis_index("i")
        left = (d + N_DEV - 1) % N_DEV
        right = (d + 1) % N_DEV

        barrier_sem = pltpu.get_barrier_semaphore()
        for nbr in (left, right):
            pl.semaphore_signal(
                barrier_sem, inc=1,
                device_id=(nbr,), device_id_type=pl.DeviceIdType.MESH,
            )
        pl.semaphore_wait(barrier_sem, 2)

        f32 = jnp.float32
        bf16 = jnp.bfloat16

        def p_left(c, j):
            return jnp.dot(a_ref[pl.ds(c * C + j * R, R), :], b_ref[:, :H],
                           preferred_element_type=f32)

        def p_right(c, j):
            return jnp.dot(a_ref[pl.ds(c * C + j * R, R), :], b_ref[:, H:],
                           preferred_element_type=f32)

        def rdma(buf_s, buf_r, slot_s, slot_r, j, ssem, rsem, dev):
            return pltpu.make_async_remote_copy(
                src_ref=buf_s.at[slot_s, pl.ds(j * R, R), :],
                dst_ref=buf_r.at[slot_r, pl.ds(j * R, R), :],
                send_sem=ssem.at[slot_s, j], recv_sem=rsem.at[slot_s, j],
                device_id=(dev,), device_id_type=pl.DeviceIdType.MESH,
            )

        rs_cw = [[rdma(cw_send, cw_recv, s, s, j, cw_ssem, cw_rsem, right)
                  for j in range(NSUB)] for s in range(N_DEV - 1)]
        rs_ccw = [[rdma(ccw_send, ccw_recv, s, s, j, ccw_ssem, ccw_rsem, left)
                   for j in range(NSUB)] for s in range(N_DEV - 1)]
        ag_cws = [[rdma(ag_cw, ag_cw, h, h + 1, j, agcw_ssem, agcw_rsem,
                        right) for j in range(NSUB)] for h in range(N_DEV - 1)]
        ag_ccws = [[rdma(ag_ccw, ag_ccw, h, h + 1, j, agccw_ssem, agccw_rsem,
                         left) for j in range(NSUB)] for h in range(N_DEV - 1)]

        c_cw0 = (d + N_DEV - 1) % N_DEV
        c_ccw0 = (d + 1) % N_DEV
        for j in range(NSUB):
            cw_send[0, pl.ds(j * R, R), :] = p_left(c_cw0, j).astype(bf16)
            rs_cw[0][j].start()
            ccw_send[0, pl.ds(j * R, R), :] = p_right(c_ccw0, j).astype(bf16)
            rs_ccw[0][j].start()

        for s in range(N_DEV - 2):
            c_cw = (d + 2 * N_DEV - 2 - s) % N_DEV
            c_ccw = (d + 2 + s) % N_DEV
            pcw = [p_left(c_cw, j) for j in range(NSUB)]
            pccw = [p_right(c_ccw, j) for j in range(NSUB)]
            for j in range(NSUB):
                sub = pl.ds(j * R, R)
                rs_cw[s][j].wait_recv()
                cw_send[s + 1, sub, :] = (
                    pcw[j] + cw_recv[s, sub, :].astype(f32)).astype(bf16)
                rs_cw[s + 1][j].start()
                rs_ccw[s][j].wait_recv()
                ccw_send[s + 1, sub, :] = (
                    pccw[j] + ccw_recv[s, sub, :].astype(f32)).astype(bf16)
                rs_ccw[s + 1][j].start()

        last = N_DEV - 2
        pf = [jnp.dot(a_ref[pl.ds(d * C + j * R, R), :], b_ref[...],
                      preferred_element_type=f32) for j in range(NSUB)]
        for j in range(NSUB):
            sub = pl.ds(j * R, R)
            rs_cw[last][j].wait_recv()
            lh = jnp.maximum(pf[j][:, :H] + cw_recv[last, sub, :].astype(f32),
                             0.0)
            ag_cw[0, sub, :] = lh.astype(bf16)
            ag_cws[0][j].start()
            rs_ccw[last][j].wait_recv()
            rh = jnp.maximum(pf[j][:, H:] + ccw_recv[last, sub, :].astype(f32),
                             0.0)
            ag_ccw[0, sub, :] = rh.astype(bf16)
            ag_ccws[0][j].start()
            out_ref[pl.ds(d * C + j * R, R), :H] = lh
            out_ref[pl.ds(d * C + j * R, R), H:] = rh

        for h in range(N_DEV - 1):
            o_cw = (d + N_DEV - 1 - h) % N_DEV
            o_ccw = (d + 1 + h) % N_DEV
            for j in range(NSUB):
                sub = pl.ds(j * R, R)
                ag_cws[h][j].wait_recv()
                if h < N_DEV - 2:
                    ag_cws[h + 1][j].start()
                ag_ccws[h][j].wait_recv()
                if h < N_DEV - 2:
                    ag_ccws[h + 1][j].start()
                out_ref[pl.ds(o_cw * C + j * R, R), :H] = (
                    ag_cw[h + 1, sub, :].astype(f32))
                out_ref[pl.ds(o_ccw * C + j * R, R), H:] = (
                    ag_ccw[h + 1, sub, :].astype(f32))

        for group in (rs_cw, rs_ccw, ag_cws, ag_ccws):
            for pair in group:
                for r in pair:
                    r.wait_send()

    sem2 = pltpu.SemaphoreType.DMA((N_DEV - 1, NSUB))
    return pl.pallas_call(
        body,
        out_shape=jax.ShapeDtypeStruct((m, n), jnp.float32),
        in_specs=[
            pl.BlockSpec(memory_space=pltpu.VMEM),
            pl.BlockSpec(memory_space=pltpu.VMEM),
        ],
        out_specs=pl.BlockSpec(memory_space=pltpu.VMEM),
        scratch_shapes=[
            pltpu.VMEM((N_DEV - 1, C, H), jnp.bfloat16),
            pltpu.VMEM((N_DEV - 1, C, H), jnp.bfloat16),
            pltpu.VMEM((N_DEV - 1, C, H), jnp.bfloat16),
            pltpu.VMEM((N_DEV - 1, C, H), jnp.bfloat16),
            pltpu.VMEM((N_DEV, C, H), jnp.bfloat16),
            pltpu.VMEM((N_DEV, C, H), jnp.bfloat16),
            sem2,
            sem2,
            sem2,
            sem2,
            sem2,
            sem2,
            sem2,
            sem2,
        ],
        compiler_params=pltpu.CompilerParams(collective_id=0),
    )(A, B)


# device time: 51486 ns/iter; 1.2315x vs baseline; 1.0237x over previous
import jax
import jax.numpy as jnp
from jax import lax
from jax.experimental import pallas as pl
from jax.experimental.pallas import tpu as pltpu

N_DEV = 4
NSUB = 3


def kernel(A, B):
    m, k = A.shape
    k2, n = B.shape
    C = m // N_DEV
    H = n // 2
    R = C // NSUB

    def body(a_ref, b_ref, out_ref,
             cw_send, cw_recv, ccw_send, ccw_recv, ag_cw, ag_ccw,
             cw_ssem, cw_rsem, ccw_ssem, ccw_rsem,
             agcw_ssem, agcw_rsem, agccw_ssem, agccw_rsem):
        d = lax.axis_index("i")
        left = (d + N_DEV - 1) % N_DEV
        right = (d + 1) % N_DEV

        barrier_sem = pltpu.get_barrier_semaphore()
        for nbr in (left, right):
            pl.semaphore_signal(
                barrier_sem, inc=1,
                device_id=(nbr,), device_id_type=pl.DeviceIdType.MESH,
            )
        pl.semaphore_wait(barrier_sem, 2)

        f32 = jnp.float32
        bf16 = jnp.bfloat16

        def p_left(c, j):
            return jnp.dot(a_ref[pl.ds(c * C + j * R, R), :], b_ref[:, :H],
                           preferred_element_type=f32)

        def p_right(c, j):
            return jnp.dot(a_ref[pl.ds(c * C + j * R, R), :], b_ref[:, H:],
                           preferred_element_type=f32)

        def rdma(buf_s, buf_r, slot_s, slot_r, j, ssem, rsem, dev):
            return pltpu.make_async_remote_copy(
                src_ref=buf_s.at[slot_s, pl.ds(j * R, R), :],
                dst_ref=buf_r.at[slot_r, pl.ds(j * R, R), :],
                send_sem=ssem.at[slot_s, j], recv_sem=rsem.at[slot_s, j],
                device_id=(dev,), device_id_type=pl.DeviceIdType.MESH,
            )

        rs_cw = [[rdma(cw_send, cw_recv, s, s, j, cw_ssem, cw_rsem, right)
                  for j in range(NSUB)] for s in range(N_DEV - 1)]
        rs_ccw = [[rdma(ccw_send, ccw_recv, s, s, j, ccw_ssem, ccw_rsem, left)
                   for j in range(NSUB)] for s in range(N_DEV - 1)]
        ag_cws = [[rdma(ag_cw, ag_cw, h, h + 1, j, agcw_ssem, agcw_rsem,
                        right) for j in range(NSUB)] for h in range(N_DEV - 1)]
        ag_ccws = [[rdma(ag_ccw, ag_ccw, h, h + 1, j, agccw_ssem, agccw_rsem,
                         left) for j in range(NSUB)] for h in range(N_DEV - 1)]

        c_cw0 = (d + N_DEV - 1) % N_DEV
        c_ccw0 = (d + 1) % N_DEV
        for j in range(NSUB):
            cw_send[0, pl.ds(j * R, R), :] = p_left(c_cw0, j).astype(bf16)
            rs_cw[0][j].start()
            ccw_send[0, pl.ds(j * R, R), :] = p_right(c_ccw0, j).astype(bf16)
            rs_ccw[0][j].start()

        for s in range(N_DEV - 2):
            c_cw = (d + 2 * N_DEV - 2 - s) % N_DEV
            c_ccw = (d + 2 + s) % N_DEV
            pcw = [p_left(c_cw, j) for j in range(NSUB)]
            pccw = [p_right(c_ccw, j) for j in range(NSUB)]
            for j in range(NSUB):
                sub = pl.ds(j * R, R)
                rs_cw[s][j].wait_recv()
                cw_send[s + 1, sub, :] = (
                    pcw[j] + cw_recv[s, sub, :].astype(f32)).astype(bf16)
                rs_cw[s + 1][j].start()
                rs_ccw[s][j].wait_recv()
                ccw_send[s + 1, sub, :] = (
                    pccw[j] + ccw_recv[s, sub, :].astype(f32)).astype(bf16)
                rs_ccw[s + 1][j].start()

        last = N_DEV - 2
        pf = [jnp.dot(a_ref[pl.ds(d * C + j * R, R), :], b_ref[...],
                      preferred_element_type=f32) for j in range(NSUB)]
        for j in range(NSUB):
            sub = pl.ds(j * R, R)
            rs_cw[last][j].wait_recv()
            lh = jnp.maximum(pf[j][:, :H] + cw_recv[last, sub, :].astype(f32),
                             0.0)
            ag_cw[0, sub, :] = lh.astype(bf16)
            ag_cws[0][j].start()
            rs_ccw[last][j].wait_recv()
            rh = jnp.maximum(pf[j][:, H:] + ccw_recv[last, sub, :].astype(f32),
                             0.0)
            ag_ccw[0, sub, :] = rh.astype(bf16)
            ag_ccws[0][j].start()
            out_ref[pl.ds(d * C + j * R, R), :H] = lh
            out_ref[pl.ds(d * C + j * R, R), H:] = rh

        for h in range(N_DEV - 1):
            o_cw = (d + N_DEV - 1 - h) % N_DEV
            o_ccw = (d + 1 + h) % N_DEV
            for j in range(NSUB):
                sub = pl.ds(j * R, R)
                ag_cws[h][j].wait_recv()
                if h < N_DEV - 2:
                    ag_cws[h + 1][j].start()
                ag_ccws[h][j].wait_recv()
                if h < N_DEV - 2:
                    ag_ccws[h + 1][j].start()
                out_ref[pl.ds(o_cw * C + j * R, R), :H] = (
                    ag_cw[h + 1, sub, :].astype(f32))
                out_ref[pl.ds(o_ccw * C + j * R, R), H:] = (
                    ag_ccw[h + 1, sub, :].astype(f32))

        for group in (rs_cw, rs_ccw, ag_cws, ag_ccws):
            for pair in group:
                for r in pair:
                    r.wait_send()

    sem2 = pltpu.SemaphoreType.DMA((N_DEV - 1, NSUB))
    return pl.pallas_call(
        body,
        out_shape=jax.ShapeDtypeStruct((m, n), jnp.float32),
        in_specs=[
            pl.BlockSpec(memory_space=pltpu.VMEM),
            pl.BlockSpec(memory_space=pltpu.VMEM),
        ],
        out_specs=pl.BlockSpec(memory_space=pltpu.VMEM),
        scratch_shapes=[
            pltpu.VMEM((N_DEV - 1, C, H), jnp.bfloat16),
            pltpu.VMEM((N_DEV - 1, C, H), jnp.bfloat16),
            pltpu.VMEM((N_DEV - 1, C, H), jnp.bfloat16),
            pltpu.VMEM((N_DEV - 1, C, H), jnp.bfloat16),
            pltpu.VMEM((N_DEV, C, H), jnp.bfloat16),
            pltpu.VMEM((N_DEV, C, H), jnp.bfloat16),
            sem2,
            sem2,
            sem2,
            sem2,
            sem2,
            sem2,
            sem2,
            sem2,
        ],
        compiler_params=pltpu.CompilerParams(collective_id=0),
    )(A, B)
